# baseline (speedup 1.0000x reference)
import functools
import math

import jax
import jax.numpy as jnp
from jax import lax
from jax.experimental import pallas as pl
from jax.experimental.pallas import tpu as pltpu
from jax.experimental.pallas import tpu_sc as plsc

N, K, IDIM, HDIM = 10000, 16, 128, 64
C1 = 64
CLAMP = 1.9

NP = 10240
NSLICE = 4
SP = NP // NSLICE
BN = 512
NB = SP // BN
NW = 32
CHUNK = 128
SROWS = K * SP
CPW = SROWS // (NW * CHUNK)
NBUF = 2
ROUNDS = CPW // NBUF - 1
PPW = SP // NW
PLOAD = PPW


def _make_sc(point_base):
    def body(table_hbm, idx_hbm, out_hbm, stage_v, idx_v, rows_v, gsem, ssem):
        wid = lax.axis_index("s") * 2 + lax.axis_index("c")

        def load_l(L, carry):
            pltpu.sync_copy(
                idx_hbm.at[pl.ds(point_base + wid * PPW + L * PLOAD, PLOAD)],
                stage_v)
            for j in range(PLOAD // 8):
                for r in range(8):
                    idx_v[L * (PLOAD // 8) + j, r * K:(r + 1) * K] = (
                        stage_v[8 * j + r, :K])
            return carry

        lax.fori_loop(0, PPW // PLOAD, load_l, 0)

        def gather(c, b):
            pltpu.async_copy(table_hbm.at[idx_v.at[c]], rows_v.at[b],
                             gsem.at[b])

        def store(chalf, b):
            row0 = (wid * (CPW // 2) + chalf) * CHUNK
            pltpu.async_copy(rows_v.at[b],
                             out_hbm.at[pl.ds(row0, CHUNK),
                                        pl.ds((b % 2) * C1, C1)],
                             ssem.at[b])

        def wait_g(b):
            pltpu.make_async_copy(table_hbm.at[idx_v.at[0]], rows_v.at[b],
                                  gsem.at[b]).wait()

        def wait_s(b):
            pltpu.make_async_copy(rows_v.at[b],
                                  out_hbm.at[pl.ds(0, CHUNK), pl.ds(0, C1)],
                                  ssem.at[b]).wait()

        for b in range(NBUF):
            gather(b, b)

        def round_body(t, carry):
            for b in range(NBUF):
                wait_g(b)
                store(t * (NBUF // 2) + b // 2, b)
            for b in range(NBUF):
                wait_s(b)
                gather((t + 1) * NBUF + b, b)
            return carry

        lax.fori_loop(0, ROUNDS, round_body, 0)

        for b in range(NBUF):
            wait_g(b)
            store(ROUNDS * (NBUF // 2) + b // 2, b)
        for b in range(NBUF):
            wait_s(b)

    return functools.partial(
        pl.kernel,
        out_type=jax.ShapeDtypeStruct((SROWS // 2, 128), jnp.float32),
        mesh=plsc.VectorSubcoreMesh(core_axis_name="c", subcore_axis_name="s"),
        scratch_types=[
            pltpu.VMEM((PLOAD, 128), jnp.int32),
            pltpu.VMEM((CPW, CHUNK), jnp.int32),
            pltpu.VMEM((NBUF, CHUNK, C1), jnp.float32),
            pltpu.SemaphoreType.DMA((NBUF,)),
            pltpu.SemaphoreType.DMA((NBUF,)),
        ],
        compiler_params=pltpu.CompilerParams(use_tc_tiling_on_sc=False),
    )(body)


_sc_gathers = [_make_sc(s * SP) for s in range(NSLICE)]


def _atan(y):
    t = jnp.abs(y)
    inv = t > 1.0
    r = jnp.where(inv, 1.0 / jnp.maximum(t, 1e-30), t)
    u = r / (1.0 + jnp.sqrt(1.0 + r * r))
    s = u * u
    p = u * (1.0 + s * (-1.0 / 3.0 + s * (1.0 / 5.0 + s * (-1.0 / 7.0 + s * (1.0 / 9.0)))))
    a = 2.0 * p
    a = jnp.where(inv, (math.pi / 2.0) - a, a)
    return jnp.sign(y) * a


def _sclamp(s):
    return CLAMP * 0.636 * _atan(s / CLAMP)


def _make_mlp_body(point_base):
    def body(x_ref, g_ref, mf_ref, bf_ref, wp_ref, b1_ref, wq_ref,
             w2_ref, b2_ref, w3_ref, b3_ref, z_ref, ld_ref):
        i = pl.program_id(0)
        f32 = jnp.float32
        z = (jnp.dot(x_ref[...], mf_ref[...], preferred_element_type=f32)
             + bf_ref[...])
        h1 = z[:, :C1]
        P = jnp.dot(h1, wp_ref[...], preferred_element_type=f32) + b1_ref[...]
        P3 = P.reshape(BN // 16, 16, 4 * HDIM)
        z3 = z.reshape(BN // 16, 16, IDIM)

        A2 = jnp.dot(g_ref[...], wq_ref[...], preferred_element_type=f32)

        n3 = (point_base + i * BN
              + 16 * lax.broadcasted_iota(jnp.int32, (BN // 16, 8, 1), 0)
              + lax.broadcasted_iota(jnp.int32, (BN // 16, 8, 1), 1))

        z_halves = []
        ld_part = jnp.zeros((1, 1), f32)
        for h in range(2):
            Ph = P3[:, 8 * h:8 * h + 8, :].reshape(BN // 2, 4 * HDIM)
            Ah = A2[:, h * 4 * HDIM:(h + 1) * 4 * HDIM]
            Ah = jnp.maximum(
                Ah.reshape(BN // 2, K, 4 * HDIM) + Ph[:, None, :], 0.0)
            Hh = jnp.maximum(
                jnp.dot(Ah.reshape(BN * K // 2, 4 * HDIM), w2_ref[...],
                        preferred_element_type=f32) + b2_ref[...], 0.0)
            acc = jnp.max(Hh.reshape(BN // 2, K, 4 * HDIM), axis=1)
            out = (jnp.dot(acc, w3_ref[...], preferred_element_type=f32)
                   + b3_ref[...])
            s1 = _sclamp(out[:, 0:64])
            t1 = out[:, 64:128]
            s2 = _sclamp(out[:, 128:192])
            t2 = out[:, 192:256]
            zh = z3[:, 8 * h:8 * h + 8, :].reshape(BN // 2, IDIM)
            h2h = zh[:, C1:]
            h2h = h2h * jnp.exp(s1) + t1
            h2h = h2h * jnp.exp(s2) + t2
            z_halves.append(
                jnp.concatenate([zh[:, :C1], h2h],
                                axis=1).reshape(BN // 16, 8, IDIM))
            valid = (n3 + 8 * h) < N
            masked = jnp.where(valid, (s1 + s2).reshape(BN // 16, 8, C1), 0.0)
            ld_part = ld_part + jnp.reshape(jnp.sum(masked), (1, 1))

        z_ref[...] = jnp.concatenate(z_halves, axis=1).reshape(BN, IDIM)

        @pl.when(i == 0)
        def _init():
            ld_ref[...] = ld_part

        @pl.when(i != 0)
        def _acc():
            ld_ref[...] = ld_ref[...] + ld_part

    return body


def _full(shape):
    return pl.BlockSpec(shape, lambda i: tuple(0 for _ in shape))


def _make_mlp(s):
    xoff = s * NB
    return pl.pallas_call(
        _make_mlp_body(s * SP),
        grid=(NB,),
        in_specs=[
            pl.BlockSpec((BN, IDIM), lambda i: (i + xoff, 0)),
            pl.BlockSpec((BN * K // 2, 2 * C1), lambda i: (i, 0)),
            _full((IDIM, IDIM)),
            _full((1, IDIM)),
            _full((C1, 4 * HDIM)),
            _full((1, 4 * HDIM)),
            _full((2 * C1, 8 * HDIM)),
            _full((4 * HDIM, 4 * HDIM)),
            _full((1, 4 * HDIM)),
            _full((4 * HDIM, 4 * HDIM)),
            _full((1, 4 * HDIM)),
        ],
        out_specs=[
            pl.BlockSpec((BN, IDIM), lambda i: (i, 0)),
            pl.BlockSpec((1, 1), lambda i: (0, 0)),
        ],
        out_shape=[
            jax.ShapeDtypeStruct((SP, IDIM), jnp.float32),
            jax.ShapeDtypeStruct((1, 1), jnp.float32),
        ],
    )


_tc_mlps = [_make_mlp(s) for s in range(NSLICE)]


def kernel(x, c, knn_idx, params):
    del c
    logs = params["actnorm_logs"].reshape(IDIM)
    biasv = params["actnorm_bias"].reshape(IDIM)
    E = jnp.exp(logs)
    E1, B1 = E[:C1], biasv[:C1]
    MF = E[None, :] * jnp.flip(jnp.eye(IDIM, dtype=jnp.float32), axis=0)

    WPs, WQfs, b1s, W2s, b2s, W3s, b3s = [], [], [], [], [], [], []
    for p in (params["c1_scale"], params["c1_shift"],
              params["c2_scale"], params["c2_shift"]):
        w1a, w1b, w1c = p["w1"][:C1], p["w1"][C1:2 * C1], p["w1"][2 * C1:]
        WQ = w1b + w1c
        WPs.append(w1a - w1b)
        WQfs.append(jnp.flip(E1[:, None] * WQ, axis=0))
        b1s.append(p["b1"] + B1 @ WQ)
        W2s.append(p["w2"]); b2s.append(p["b2"])
        W3s.append(p["w3"]); b3s.append(p["b3"])
    WP = jnp.concatenate(WPs, axis=1)
    WQf = jnp.concatenate(WQfs, axis=1)
    eye2 = jnp.eye(2, dtype=jnp.float32)
    WQ2 = (eye2[:, None, :, None]
           * jnp.stack([WQf, WQf])[:, :, None, :]).reshape(2 * C1, 8 * HDIM)
    eye4 = jnp.eye(4, dtype=jnp.float32)
    W2BD = (eye4[:, None, :, None]
            * jnp.stack(W2s)[:, :, None, :]).reshape(4 * HDIM, 4 * HDIM)
    W3BD = (eye4[:, None, :, None]
            * jnp.stack(W3s)[:, :, None, :]).reshape(4 * HDIM, 4 * HDIM)
    b1cat = jnp.concatenate(b1s)[None, :]
    b2cat = jnp.concatenate(b2s)[None, :]
    b3cat = jnp.concatenate(b3s)[None, :]

    xs = x[0, :, C1:]
    idxp = jnp.pad(knn_idx[0].astype(jnp.int32), ((0, NP - N), (0, 128 - K)))
    x_pad = jnp.pad(x[0], ((0, NP - N), (0, 0)))

    zs, ld_total = [], jnp.sum(logs) * N
    for s in range(NSLICE):
        G = _sc_gathers[s](xs, idxp)
        zout, ld = _tc_mlps[s](x_pad, G, MF, biasv[None, :], WP, b1cat, WQ2,
                               W2BD, b2cat, W3BD, b3cat)
        zs.append(zout)
        ld_total = ld_total + ld[0, 0]
    z = jnp.concatenate(zs, axis=0)[:N].reshape(1, N, IDIM)
    return z, jnp.reshape(ld_total, (1,))


# speedup vs baseline: 1.4147x; 1.4147x over previous
import functools
import math

import jax
import jax.numpy as jnp
from jax import lax
from jax.experimental import pallas as pl
from jax.experimental.pallas import tpu as pltpu
from jax.experimental.pallas import tpu_sc as plsc

N, K, IDIM, HDIM = 10000, 16, 128, 64
C1 = 64
CLAMP = 1.9

NP = 10240
NSLICE = 2
SP = NP // NSLICE
BN = 1024
NB = SP // BN
NW = 32
CHUNK = 128
SROWS = K * SP
CPW = SROWS // (NW * CHUNK)
NBUF = 4
ROUNDS = CPW // NBUF - 1
PPW = SP // NW
PLOAD = PPW // 2


def _make_sc(point_base):
    def body(table_hbm, idx_hbm, out_hbm, stage_v, idx_v, rows_v, table_sh,
             gsem, ssem):
        sid = lax.axis_index("s")
        wid = sid * 2 + lax.axis_index("c")

        @pl.when(sid == 0)
        def _load_table():
            pltpu.sync_copy(table_hbm, table_sh)

        plsc.subcore_barrier()

        def load_l(L, carry):
            pltpu.sync_copy(
                idx_hbm.at[pl.ds(point_base + wid * PPW + L * PLOAD, PLOAD)],
                stage_v)
            for j in range(PLOAD // 8):
                for r in range(8):
                    idx_v[L * (PLOAD // 8) + j, r * K:(r + 1) * K] = (
                        stage_v[8 * j + r, :K])
            return carry

        lax.fori_loop(0, PPW // PLOAD, load_l, 0)

        def gather(c, b):
            pltpu.async_copy(table_sh.at[idx_v.at[c]], rows_v.at[b],
                             gsem.at[b])

        def store(chalf, b):
            row0 = (wid * (CPW // 2) + chalf) * CHUNK
            pltpu.async_copy(rows_v.at[b],
                             out_hbm.at[pl.ds(row0, CHUNK),
                                        pl.ds((b % 2) * C1, C1)],
                             ssem.at[b])

        def wait_g(b):
            pltpu.make_async_copy(table_sh.at[idx_v.at[0]], rows_v.at[b],
                                  gsem.at[b]).wait()

        def wait_s(b):
            pltpu.make_async_copy(rows_v.at[b],
                                  out_hbm.at[pl.ds(0, CHUNK), pl.ds(0, C1)],
                                  ssem.at[b]).wait()

        for b in range(NBUF):
            gather(b, b)

        def round_body(t, carry):
            for b in range(NBUF):
                wait_g(b)
                store(t * (NBUF // 2) + b // 2, b)
            for b in range(NBUF):
                wait_s(b)
                gather((t + 1) * NBUF + b, b)
            return carry

        lax.fori_loop(0, ROUNDS, round_body, 0)

        for b in range(NBUF):
            wait_g(b)
            store(ROUNDS * (NBUF // 2) + b // 2, b)
        for b in range(NBUF):
            wait_s(b)

    return functools.partial(
        pl.kernel,
        out_type=jax.ShapeDtypeStruct((SROWS // 2, 128), jnp.float32),
        mesh=plsc.VectorSubcoreMesh(core_axis_name="c", subcore_axis_name="s"),
        scratch_types=[
            pltpu.VMEM((PLOAD, 128), jnp.int32),
            pltpu.VMEM((CPW, CHUNK), jnp.int32),
            pltpu.VMEM((NBUF, CHUNK, C1), jnp.float32),
            pltpu.VMEM_SHARED((N, C1), jnp.float32),
            pltpu.SemaphoreType.DMA((NBUF,)),
            pltpu.SemaphoreType.DMA((NBUF,)),
        ],
        compiler_params=pltpu.CompilerParams(use_tc_tiling_on_sc=False),
    )(body)


_sc_gathers = [_make_sc(s * SP) for s in range(NSLICE)]


def _atan(y):
    t = jnp.abs(y)
    inv = t > 1.0
    r = jnp.where(inv, 1.0 / jnp.maximum(t, 1e-30), t)
    u = r / (1.0 + jnp.sqrt(1.0 + r * r))
    s = u * u
    p = u * (1.0 + s * (-1.0 / 3.0 + s * (1.0 / 5.0 + s * (-1.0 / 7.0 + s * (1.0 / 9.0)))))
    a = 2.0 * p
    a = jnp.where(inv, (math.pi / 2.0) - a, a)
    return jnp.sign(y) * a


def _sclamp(s):
    return CLAMP * 0.636 * _atan(s / CLAMP)


def _make_mlp_body(point_base):
    def body(x_ref, g_ref, mf_ref, bf_ref, wp_ref, b1_ref, wq_ref,
             w2_ref, b2_ref, w3_ref, b3_ref, z_ref, ld_ref):
        i = pl.program_id(0)
        f32 = jnp.float32
        z = (jnp.dot(x_ref[...], mf_ref[...], preferred_element_type=f32)
             + bf_ref[...])
        h1 = z[:, :C1]
        P = jnp.dot(h1, wp_ref[...], preferred_element_type=f32) + b1_ref[...]
        P3 = P.reshape(BN // 16, 16, 4 * HDIM)
        z3 = z.reshape(BN // 16, 16, IDIM)

        A2 = jnp.dot(g_ref[...], wq_ref[...], preferred_element_type=f32)

        n3 = (point_base + i * BN
              + 16 * lax.broadcasted_iota(jnp.int32, (BN // 16, 8, 1), 0)
              + lax.broadcasted_iota(jnp.int32, (BN // 16, 8, 1), 1))

        z_halves = []
        ld_part = jnp.zeros((1, 1), f32)
        for h in range(2):
            Ph = P3[:, 8 * h:8 * h + 8, :].reshape(BN // 2, 4 * HDIM)
            Ah = A2[:, h * 4 * HDIM:(h + 1) * 4 * HDIM]
            Ah = jnp.maximum(
                Ah.reshape(BN // 2, K, 4 * HDIM) + Ph[:, None, :], 0.0)
            Hh = jnp.maximum(
                jnp.dot(Ah.reshape(BN * K // 2, 4 * HDIM), w2_ref[...],
                        preferred_element_type=f32) + b2_ref[...], 0.0)
            acc = jnp.max(Hh.reshape(BN // 2, K, 4 * HDIM), axis=1)
            out = (jnp.dot(acc, w3_ref[...], preferred_element_type=f32)
                   + b3_ref[...])
            s1 = _sclamp(out[:, 0:64])
            t1 = out[:, 64:128]
            s2 = _sclamp(out[:, 128:192])
            t2 = out[:, 192:256]
            zh = z3[:, 8 * h:8 * h + 8, :].reshape(BN // 2, IDIM)
            h2h = zh[:, C1:]
            h2h = h2h * jnp.exp(s1) + t1
            h2h = h2h * jnp.exp(s2) + t2
            z_halves.append(
                jnp.concatenate([zh[:, :C1], h2h],
                                axis=1).reshape(BN // 16, 8, IDIM))
            valid = (n3 + 8 * h) < N
            masked = jnp.where(valid, (s1 + s2).reshape(BN // 16, 8, C1), 0.0)
            ld_part = ld_part + jnp.reshape(jnp.sum(masked), (1, 1))

        z_ref[...] = jnp.concatenate(z_halves, axis=1).reshape(BN, IDIM)

        @pl.when(i == 0)
        def _init():
            ld_ref[...] = ld_part

        @pl.when(i != 0)
        def _acc():
            ld_ref[...] = ld_ref[...] + ld_part

    return body


def _full(shape):
    return pl.BlockSpec(shape, lambda i: tuple(0 for _ in shape))


def _make_mlp(s):
    xoff = s * NB
    return pl.pallas_call(
        _make_mlp_body(s * SP),
        grid=(NB,),
        in_specs=[
            pl.BlockSpec((BN, IDIM), lambda i: (i + xoff, 0)),
            pl.BlockSpec((BN * K // 2, 2 * C1), lambda i: (i, 0)),
            _full((IDIM, IDIM)),
            _full((1, IDIM)),
            _full((C1, 4 * HDIM)),
            _full((1, 4 * HDIM)),
            _full((2 * C1, 8 * HDIM)),
            _full((4 * HDIM, 4 * HDIM)),
            _full((1, 4 * HDIM)),
            _full((4 * HDIM, 4 * HDIM)),
            _full((1, 4 * HDIM)),
        ],
        out_specs=[
            pl.BlockSpec((BN, IDIM), lambda i: (i, 0)),
            pl.BlockSpec((1, 1), lambda i: (0, 0)),
        ],
        out_shape=[
            jax.ShapeDtypeStruct((SP, IDIM), jnp.float32),
            jax.ShapeDtypeStruct((1, 1), jnp.float32),
        ],
    )


_tc_mlps = [_make_mlp(s) for s in range(NSLICE)]


def kernel(x, c, knn_idx, params):
    del c
    logs = params["actnorm_logs"].reshape(IDIM)
    biasv = params["actnorm_bias"].reshape(IDIM)
    E = jnp.exp(logs)
    E1, B1 = E[:C1], biasv[:C1]
    MF = E[None, :] * jnp.flip(jnp.eye(IDIM, dtype=jnp.float32), axis=0)

    WPs, WQfs, b1s, W2s, b2s, W3s, b3s = [], [], [], [], [], [], []
    for p in (params["c1_scale"], params["c1_shift"],
              params["c2_scale"], params["c2_shift"]):
        w1a, w1b, w1c = p["w1"][:C1], p["w1"][C1:2 * C1], p["w1"][2 * C1:]
        WQ = w1b + w1c
        WPs.append(w1a - w1b)
        WQfs.append(jnp.flip(E1[:, None] * WQ, axis=0))
        b1s.append(p["b1"] + B1 @ WQ)
        W2s.append(p["w2"]); b2s.append(p["b2"])
        W3s.append(p["w3"]); b3s.append(p["b3"])
    WP = jnp.concatenate(WPs, axis=1)
    WQf = jnp.concatenate(WQfs, axis=1)
    eye2 = jnp.eye(2, dtype=jnp.float32)
    WQ2 = (eye2[:, None, :, None]
           * jnp.stack([WQf, WQf])[:, :, None, :]).reshape(2 * C1, 8 * HDIM)
    eye4 = jnp.eye(4, dtype=jnp.float32)
    W2BD = (eye4[:, None, :, None]
            * jnp.stack(W2s)[:, :, None, :]).reshape(4 * HDIM, 4 * HDIM)
    W3BD = (eye4[:, None, :, None]
            * jnp.stack(W3s)[:, :, None, :]).reshape(4 * HDIM, 4 * HDIM)
    b1cat = jnp.concatenate(b1s)[None, :]
    b2cat = jnp.concatenate(b2s)[None, :]
    b3cat = jnp.concatenate(b3s)[None, :]

    xs = x[0, :, C1:]
    idxp = jnp.pad(knn_idx[0].astype(jnp.int32), ((0, NP - N), (0, 128 - K)))
    x_pad = jnp.pad(x[0], ((0, NP - N), (0, 0)))

    zs, ld_total = [], jnp.sum(logs) * N
    for s in range(NSLICE):
        G = _sc_gathers[s](xs, idxp)
        zout, ld = _tc_mlps[s](x_pad, G, MF, biasv[None, :], WP, b1cat, WQ2,
                               W2BD, b2cat, W3BD, b3cat)
        zs.append(zout)
        ld_total = ld_total + ld[0, 0]
    z = jnp.concatenate(zs, axis=0)[:N].reshape(1, N, IDIM)
    return z, jnp.reshape(ld_total, (1,))
